# Initial kernel scaffold; baseline (speedup 1.0000x reference)
import jax
import jax.numpy as jnp
from jax.experimental import pallas as pl


def kernel(input, G_vals, L_vals, F2V_vals, NS, EW, coeffs, G_rows, G_cols, L_rows, L_cols, F2V_rows, F2V_cols):
    raise NotImplementedError("write your pallas kernel here")


# speedup vs baseline: 46.8225x; 46.8225x over previous
import functools

import jax
import jax.numpy as jnp
from jax import lax
from jax.experimental import pallas as pl
from jax.experimental.pallas import tpu as pltpu
from jax.experimental.pallas import tpu_sc as plsc

NVK = 40962
NFK = 81920
BC = 256
NW = 32
NVP = 41472

FC = 8
FPW = NFK // NW
NCH_A = FPW // FC

VC = 8
VPW = NVP // NW
NCH_B = VPW // VC

_MESH = plsc.VectorSubcoreMesh(core_axis_name="c", subcore_axis_name="s")


def _widx(w):
    return jnp.full((16,), w, dtype=jnp.float32)


@functools.partial(
    pl.kernel,
    out_type=jax.ShapeDtypeStruct((NFK, 2 * BC), jnp.float32),
    mesh=_MESH,
    scratch_types=[
        pltpu.VMEM((FC * 9,), jnp.int32),
        pltpu.VMEM((FC * 9 + 16,), jnp.float32),
        pltpu.VMEM((FC * 3 + 16,), jnp.float32),
        pltpu.VMEM((FC * 3 + 16,), jnp.float32),
        pltpu.VMEM((FC * 9, BC), jnp.float32),
        pltpu.VMEM((FC, 2 * BC), jnp.float32),
        pltpu.SemaphoreType.DMA,
    ],
)
def _gf_kernel(xt_hbm, idx_hbm, gv_hbm, ew_hbm, ns_hbm, gf_hbm,
               idx_v, gv_v, ew_v, ns_v, rows_v, out_v, sem):
    wid = lax.axis_index("s") * 2 + lax.axis_index("c")
    base = wid * FPW

    def chunk_body(cix, carry):
        f0 = base + cix * FC
        pltpu.sync_copy(idx_hbm.at[pl.ds(f0 * 9, FC * 9)], idx_v)
        pltpu.sync_copy(gv_hbm.at[pl.ds(f0 * 9, FC * 9)], gv_v.at[pl.ds(0, FC * 9)])
        pltpu.sync_copy(ew_hbm.at[pl.ds(f0 * 3, FC * 3)], ew_v.at[pl.ds(0, FC * 3)])
        pltpu.sync_copy(ns_hbm.at[pl.ds(f0 * 3, FC * 3)], ns_v.at[pl.ds(0, FC * 3)])
        pltpu.async_copy(xt_hbm.at[idx_v], rows_v, sem).wait()

        def face_body(i, fc):
            acc_e = [jnp.zeros((16,), jnp.float32) for _ in range(16)]
            acc_n = [jnp.zeros((16,), jnp.float32) for _ in range(16)]
            wg = gv_v[pl.ds(9 * i, 16)]
            we = ew_v[pl.ds(3 * i, 16)]
            wn = ns_v[pl.ds(3 * i, 16)]
            for j in range(9):
                k = j // 3
                wev = _widx(wg[j] * we[k])
                wnv = _widx(wg[j] * wn[k])
                for c in range(16):
                    r = rows_v[9 * i + j, pl.ds(16 * c, 16)]
                    acc_e[c] = acc_e[c] + wev * r
                    acc_n[c] = acc_n[c] + wnv * r
            for c in range(16):
                out_v[i, pl.ds(16 * c, 16)] = acc_e[c]
                out_v[i, pl.ds(BC + 16 * c, 16)] = acc_n[c]
            return fc

        lax.fori_loop(0, FC, face_body, 0)
        pltpu.sync_copy(out_v, gf_hbm.at[pl.ds(f0, FC)])
        return carry

    lax.fori_loop(0, NCH_A, chunk_body, 0)


@functools.partial(
    pl.kernel,
    out_type=(
        jax.ShapeDtypeStruct((NVP, BC), jnp.float32),
        jax.ShapeDtypeStruct((NVP, BC), jnp.float32),
        jax.ShapeDtypeStruct((NVP, BC), jnp.float32),
    ),
    mesh=_MESH,
    scratch_types=[
        pltpu.VMEM((VC * 7,), jnp.int32),
        pltpu.VMEM((VC * 7 + 16,), jnp.float32),
        pltpu.VMEM((VC * 6,), jnp.int32),
        pltpu.VMEM((VC * 6 + 16,), jnp.float32),
        pltpu.VMEM((VC * 7, BC), jnp.float32),
        pltpu.VMEM((VC * 6, 2 * BC), jnp.float32),
        pltpu.VMEM((VC, BC), jnp.float32),
        pltpu.VMEM((VC, BC), jnp.float32),
        pltpu.VMEM((VC, BC), jnp.float32),
        pltpu.SemaphoreType.DMA,
        pltpu.SemaphoreType.DMA,
    ],
)
def _lv_kernel(xt_hbm, gf_hbm, li_hbm, lv_hbm, fi_hbm, fv_hbm,
               lap_hbm, gve_hbm, gvn_hbm,
               li_v, lv_v, fi_v, fv_v, rx_v, rg_v,
               ol_v, oe_v, on_v, sem_x, sem_g):
    wid = lax.axis_index("s") * 2 + lax.axis_index("c")
    base = wid * VPW

    def chunk_body(cix, carry):
        v0 = base + cix * VC
        pltpu.sync_copy(li_hbm.at[pl.ds(v0 * 7, VC * 7)], li_v)
        pltpu.sync_copy(lv_hbm.at[pl.ds(v0 * 7, VC * 7)], lv_v.at[pl.ds(0, VC * 7)])
        pltpu.sync_copy(fi_hbm.at[pl.ds(v0 * 6, VC * 6)], fi_v)
        pltpu.sync_copy(fv_hbm.at[pl.ds(v0 * 6, VC * 6)], fv_v.at[pl.ds(0, VC * 6)])
        cx = pltpu.async_copy(xt_hbm.at[li_v], rx_v, sem_x)
        cg = pltpu.async_copy(gf_hbm.at[fi_v], rg_v, sem_g)
        cx.wait()
        cg.wait()

        def vert_body(i, fc):
            acc_l = [jnp.zeros((16,), jnp.float32) for _ in range(16)]
            wlv = lv_v[pl.ds(7 * i, 16)]
            for j in range(7):
                wl = _widx(wlv[j])
                for c in range(16):
                    r = rx_v[7 * i + j, pl.ds(16 * c, 16)]
                    acc_l[c] = acc_l[c] + wl * r
            for c in range(16):
                ol_v[i, pl.ds(16 * c, 16)] = acc_l[c]
            acc_e = [jnp.zeros((16,), jnp.float32) for _ in range(16)]
            acc_n = [jnp.zeros((16,), jnp.float32) for _ in range(16)]
            wfv = fv_v[pl.ds(6 * i, 16)]
            for j in range(6):
                wf = _widx(wfv[j])
                for c in range(16):
                    re = rg_v[6 * i + j, pl.ds(16 * c, 16)]
                    rn = rg_v[6 * i + j, pl.ds(BC + 16 * c, 16)]
                    acc_e[c] = acc_e[c] + wf * re
                    acc_n[c] = acc_n[c] + wf * rn
            for c in range(16):
                oe_v[i, pl.ds(16 * c, 16)] = acc_e[c]
                on_v[i, pl.ds(16 * c, 16)] = acc_n[c]
            return fc

        lax.fori_loop(0, VC, vert_body, 0)
        pltpu.sync_copy(ol_v, lap_hbm.at[pl.ds(v0, VC)])
        pltpu.sync_copy(oe_v, gve_hbm.at[pl.ds(v0, VC)])
        pltpu.sync_copy(on_v, gvn_hbm.at[pl.ds(v0, VC)])
        return carry

    lax.fori_loop(0, NCH_B, chunk_body, 0)


def _mm_body(x_ref, l_ref, e_ref, n_ref, cs_ref, o_ref):
    cs = cs_ref[...]
    acc = jnp.dot(x_ref[...], cs[0], preferred_element_type=jnp.float32)
    acc += jnp.dot(l_ref[...], cs[1], preferred_element_type=jnp.float32)
    acc += jnp.dot(e_ref[...], cs[2], preferred_element_type=jnp.float32)
    acc += jnp.dot(n_ref[...], cs[3], preferred_element_type=jnp.float32)
    o_ref[...] = acc


_ROWS_PER_BLK = 2048
_MM_GRID = NVP * 4 // _ROWS_PER_BLK

_mm = pl.pallas_call(
    _mm_body,
    grid=(_MM_GRID,),
    in_specs=[
        pl.BlockSpec((_ROWS_PER_BLK, 64), lambda i: (i, 0)),
        pl.BlockSpec((_ROWS_PER_BLK, 64), lambda i: (i, 0)),
        pl.BlockSpec((_ROWS_PER_BLK, 64), lambda i: (i, 0)),
        pl.BlockSpec((_ROWS_PER_BLK, 64), lambda i: (i, 0)),
        pl.BlockSpec((4, 64, 64), lambda i: (0, 0, 0)),
    ],
    out_specs=pl.BlockSpec((_ROWS_PER_BLK, 64), lambda i: (i, 0)),
    out_shape=jax.ShapeDtypeStruct((NVP * 4, 64), jnp.float32),
)


def kernel(input, G_vals, L_vals, F2V_vals, NS, EW, coeffs,
           G_rows, G_cols, L_rows, L_cols, F2V_rows, F2V_cols):
    pad = NVP - NVK
    xt = input.transpose(2, 0, 1).reshape(NVK, BC)
    xtp = jnp.pad(xt, ((0, pad), (0, 0)))
    idx9 = G_cols.reshape(3, NFK, 3).transpose(1, 0, 2).reshape(-1)
    gv9 = G_vals.reshape(3, NFK, 3).transpose(1, 0, 2).reshape(-1)
    ew3 = EW.reshape(-1)
    ns3 = NS.reshape(-1)
    li7 = jnp.pad(L_cols.reshape(NVK, 7), ((0, pad), (0, 0))).reshape(-1)
    lv7 = jnp.pad(L_vals.reshape(NVK, 7), ((0, pad), (0, 0))).reshape(-1)
    fi6 = jnp.pad(F2V_cols.reshape(NVK, 6), ((0, pad), (0, 0))).reshape(-1)
    fv6 = jnp.pad(F2V_vals.reshape(NVK, 6), ((0, pad), (0, 0))).reshape(-1)
    cs = jnp.stack([coeffs[k::4] for k in range(4)])

    gf = _gf_kernel(xtp, idx9, gv9, ew3, ns3)
    lap, gve, gvn = _lv_kernel(xtp, gf, li7, lv7, fi6, fv6)

    y4 = _mm(xtp.reshape(NVP * 4, 64), lap.reshape(NVP * 4, 64),
             gve.reshape(NVP * 4, 64), gvn.reshape(NVP * 4, 64), cs)
    return y4.reshape(NVP, 4, 64)[:NVK].transpose(1, 2, 0)


# speedup vs baseline: 75.4700x; 1.6118x over previous
import functools

import jax
import jax.numpy as jnp
from jax import lax
from jax.experimental import pallas as pl
from jax.experimental.pallas import tpu as pltpu
from jax.experimental.pallas import tpu_sc as plsc

NVK = 40962
NFK = 81920
BC = 256
NW = 32
NVP = 41472

FC = 8
FPW = NFK // NW
NCH_A = FPW // FC

VC = 8
VPW = NVP // NW
NCH_B = VPW // VC

_MESH = plsc.VectorSubcoreMesh(core_axis_name="c", subcore_axis_name="s")


def _widx(w):
    return jnp.full((16,), w, dtype=jnp.float32)


@functools.partial(
    pl.kernel,
    out_type=jax.ShapeDtypeStruct((NFK, 2 * BC), jnp.float32),
    mesh=_MESH,
    scratch_types=[
        pltpu.VMEM((2, FC * 9), jnp.int32),
        pltpu.VMEM((2, FC, 16), jnp.float32),
        pltpu.VMEM((2, FC * 9, BC), jnp.float32),
        pltpu.VMEM((2, FC, 2 * BC), jnp.float32),
        pltpu.SemaphoreType.DMA,
        pltpu.SemaphoreType.DMA,
        pltpu.SemaphoreType.DMA,
        pltpu.SemaphoreType.DMA,
        pltpu.SemaphoreType.DMA,
        pltpu.SemaphoreType.DMA,
    ],
)
def _gf_kernel(xt_hbm, idx_hbm, wm_hbm, gf_hbm,
               idx_v, wm_v, rows_v, out_v,
               ss0, ss1, gs0, gs1, os0, os1):
    wid = lax.axis_index("s") * 2 + lax.axis_index("c")
    base = wid * FPW
    ss = (ss0, ss1)
    gs = (gs0, gs1)
    osm = (os0, os1)

    def slab_issue(c, p):
        f0 = base + c * FC
        pltpu.async_copy(idx_hbm.at[pl.ds(f0 * 9, FC * 9)], idx_v.at[p], ss[p])
        pltpu.async_copy(wm_hbm.at[pl.ds(f0, FC)], wm_v.at[p], ss[p])

    def slab_wait(c, p):
        f0 = base + c * FC
        pltpu.make_async_copy(idx_hbm.at[pl.ds(f0 * 9, FC * 9)], idx_v.at[p], ss[p]).wait()
        pltpu.make_async_copy(wm_hbm.at[pl.ds(f0, FC)], wm_v.at[p], ss[p]).wait()

    def gather_issue(p):
        pltpu.async_copy(xt_hbm.at[idx_v.at[p]], rows_v.at[p], gs[p])

    def gather_wait(p):
        pltpu.make_async_copy(xt_hbm.at[idx_v.at[p]], rows_v.at[p], gs[p]).wait()

    def out_issue(c, p):
        f0 = base + c * FC
        pltpu.async_copy(out_v.at[p], gf_hbm.at[pl.ds(f0, FC)], osm[p])

    def out_wait(c, p):
        f0 = base + c * FC
        pltpu.make_async_copy(out_v.at[p], gf_hbm.at[pl.ds(f0, FC)], osm[p]).wait()

    slab_issue(0, 0)
    slab_wait(0, 0)
    gather_issue(0)
    slab_issue(1, 1)

    def body2(c2, carry):
        for p in (0, 1):
            c = 2 * c2 + p
            q = 1 - p
            gather_wait(p)

            @pl.when(c + 1 < NCH_A)
            def _():
                slab_wait(c + 1, q)
                gather_issue(q)

            @pl.when(c >= 2)
            def _():
                out_wait(c, p)

            def face_body(i, fc):
                wrow = wm_v[p, i, :]
                acc_e = [jnp.zeros((16,), jnp.float32) for _ in range(16)]
                acc_n = [jnp.zeros((16,), jnp.float32) for _ in range(16)]
                for j in range(9):
                    k = j // 3
                    wev = _widx(wrow[j] * wrow[9 + k])
                    wnv = _widx(wrow[j] * wrow[12 + k])
                    for cc in range(16):
                        r = rows_v[p, 9 * i + j, pl.ds(16 * cc, 16)]
                        acc_e[cc] = acc_e[cc] + wev * r
                        acc_n[cc] = acc_n[cc] + wnv * r
                for cc in range(16):
                    out_v[p, i, pl.ds(16 * cc, 16)] = acc_e[cc]
                    out_v[p, i, pl.ds(BC + 16 * cc, 16)] = acc_n[cc]
                return fc

            lax.fori_loop(0, FC, face_body, 0)
            out_issue(c, p)

            @pl.when(c + 2 < NCH_A)
            def _():
                slab_issue(c + 2, p)
        return carry

    lax.fori_loop(0, NCH_A // 2, body2, 0)
    out_wait(NCH_A - 2, 0)
    out_wait(NCH_A - 1, 1)


@functools.partial(
    pl.kernel,
    out_type=(
        jax.ShapeDtypeStruct((NVP, BC), jnp.float32),
        jax.ShapeDtypeStruct((NVP, BC), jnp.float32),
        jax.ShapeDtypeStruct((NVP, BC), jnp.float32),
    ),
    mesh=_MESH,
    scratch_types=[
        pltpu.VMEM((2, VC * 7), jnp.int32),
        pltpu.VMEM((2, VC * 6), jnp.int32),
        pltpu.VMEM((2, VC, 16), jnp.float32),
        pltpu.VMEM((2, VC * 7, BC), jnp.float32),
        pltpu.VMEM((2, VC * 6, 2 * BC), jnp.float32),
        pltpu.VMEM((2, VC, BC), jnp.float32),
        pltpu.VMEM((2, VC, BC), jnp.float32),
        pltpu.VMEM((2, VC, BC), jnp.float32),
        pltpu.SemaphoreType.DMA,
        pltpu.SemaphoreType.DMA,
        pltpu.SemaphoreType.DMA,
        pltpu.SemaphoreType.DMA,
        pltpu.SemaphoreType.DMA,
        pltpu.SemaphoreType.DMA,
    ],
)
def _lv_kernel(xt_hbm, gf_hbm, li_hbm, fi_hbm, wb_hbm,
               lap_hbm, gve_hbm, gvn_hbm,
               li_v, fi_v, wb_v, rx_v, rg_v, ol_v, oe_v, on_v,
               ss0, ss1, gs0, gs1, os0, os1):
    wid = lax.axis_index("s") * 2 + lax.axis_index("c")
    base = wid * VPW
    ss = (ss0, ss1)
    gs = (gs0, gs1)
    osm = (os0, os1)

    def slab_issue(c, p):
        v0 = base + c * VC
        pltpu.async_copy(li_hbm.at[pl.ds(v0 * 7, VC * 7)], li_v.at[p], ss[p])
        pltpu.async_copy(fi_hbm.at[pl.ds(v0 * 6, VC * 6)], fi_v.at[p], ss[p])
        pltpu.async_copy(wb_hbm.at[pl.ds(v0, VC)], wb_v.at[p], ss[p])

    def slab_wait(c, p):
        v0 = base + c * VC
        pltpu.make_async_copy(li_hbm.at[pl.ds(v0 * 7, VC * 7)], li_v.at[p], ss[p]).wait()
        pltpu.make_async_copy(fi_hbm.at[pl.ds(v0 * 6, VC * 6)], fi_v.at[p], ss[p]).wait()
        pltpu.make_async_copy(wb_hbm.at[pl.ds(v0, VC)], wb_v.at[p], ss[p]).wait()

    def gather_issue(p):
        pltpu.async_copy(xt_hbm.at[li_v.at[p]], rx_v.at[p], gs[p])
        pltpu.async_copy(gf_hbm.at[fi_v.at[p]], rg_v.at[p], gs[p])

    def gather_wait(p):
        pltpu.make_async_copy(xt_hbm.at[li_v.at[p]], rx_v.at[p], gs[p]).wait()
        pltpu.make_async_copy(gf_hbm.at[fi_v.at[p]], rg_v.at[p], gs[p]).wait()

    def out_issue(c, p):
        v0 = base + c * VC
        pltpu.async_copy(ol_v.at[p], lap_hbm.at[pl.ds(v0, VC)], osm[p])
        pltpu.async_copy(oe_v.at[p], gve_hbm.at[pl.ds(v0, VC)], osm[p])
        pltpu.async_copy(on_v.at[p], gvn_hbm.at[pl.ds(v0, VC)], osm[p])

    def out_wait(c, p):
        v0 = base + c * VC
        pltpu.make_async_copy(ol_v.at[p], lap_hbm.at[pl.ds(v0, VC)], osm[p]).wait()
        pltpu.make_async_copy(oe_v.at[p], gve_hbm.at[pl.ds(v0, VC)], osm[p]).wait()
        pltpu.make_async_copy(on_v.at[p], gvn_hbm.at[pl.ds(v0, VC)], osm[p]).wait()

    slab_issue(0, 0)
    slab_wait(0, 0)
    gather_issue(0)
    slab_issue(1, 1)

    def body2(c2, carry):
        for p in (0, 1):
            c = 2 * c2 + p
            q = 1 - p
            gather_wait(p)

            @pl.when(c + 1 < NCH_B)
            def _():
                slab_wait(c + 1, q)
                gather_issue(q)

            @pl.when(c >= 2)
            def _():
                out_wait(c, p)

            def vert_body(i, fc):
                wrow = wb_v[p, i, :]
                acc_l = [jnp.zeros((16,), jnp.float32) for _ in range(16)]
                for j in range(7):
                    wl = _widx(wrow[j])
                    for cc in range(16):
                        r = rx_v[p, 7 * i + j, pl.ds(16 * cc, 16)]
                        acc_l[cc] = acc_l[cc] + wl * r
                for cc in range(16):
                    ol_v[p, i, pl.ds(16 * cc, 16)] = acc_l[cc]
                acc_e = [jnp.zeros((16,), jnp.float32) for _ in range(16)]
                acc_n = [jnp.zeros((16,), jnp.float32) for _ in range(16)]
                for j in range(6):
                    wf = _widx(wrow[7 + j])
                    for cc in range(16):
                        re = rg_v[p, 6 * i + j, pl.ds(16 * cc, 16)]
                        rn = rg_v[p, 6 * i + j, pl.ds(BC + 16 * cc, 16)]
                        acc_e[cc] = acc_e[cc] + wf * re
                        acc_n[cc] = acc_n[cc] + wf * rn
                for cc in range(16):
                    oe_v[p, i, pl.ds(16 * cc, 16)] = acc_e[cc]
                    on_v[p, i, pl.ds(16 * cc, 16)] = acc_n[cc]
                return fc

            lax.fori_loop(0, VC, vert_body, 0)
            out_issue(c, p)

            @pl.when(c + 2 < NCH_B)
            def _():
                slab_issue(c + 2, p)
        return carry

    lax.fori_loop(0, NCH_B // 2, body2, 0)
    out_wait(NCH_B - 2, 0)
    out_wait(NCH_B - 1, 1)


def _tin_body(x_ref, o_ref):
    o_ref[...] = x_ref[...].T


_tin = pl.pallas_call(
    _tin_body,
    grid=(NVP // 512,),
    in_specs=[pl.BlockSpec((BC, 512), lambda i: (0, i))],
    out_specs=pl.BlockSpec((512, BC), lambda i: (i, 0)),
    out_shape=jax.ShapeDtypeStruct((NVP, BC), jnp.float32),
)


def _mm_body(x_ref, l_ref, e_ref, n_ref, cs_ref, o_ref):
    cs = cs_ref[...]
    acc = jnp.dot(x_ref[...], cs[0], preferred_element_type=jnp.float32)
    acc += jnp.dot(l_ref[...], cs[1], preferred_element_type=jnp.float32)
    acc += jnp.dot(e_ref[...], cs[2], preferred_element_type=jnp.float32)
    acc += jnp.dot(n_ref[...], cs[3], preferred_element_type=jnp.float32)
    o_ref[...] = acc


_ROWS_PER_BLK = 2048
_MM_GRID = NVP * 4 // _ROWS_PER_BLK

_mm = pl.pallas_call(
    _mm_body,
    grid=(_MM_GRID,),
    in_specs=[
        pl.BlockSpec((_ROWS_PER_BLK, 64), lambda i: (i, 0)),
        pl.BlockSpec((_ROWS_PER_BLK, 64), lambda i: (i, 0)),
        pl.BlockSpec((_ROWS_PER_BLK, 64), lambda i: (i, 0)),
        pl.BlockSpec((_ROWS_PER_BLK, 64), lambda i: (i, 0)),
        pl.BlockSpec((4, 64, 64), lambda i: (0, 0, 0)),
    ],
    out_specs=pl.BlockSpec((_ROWS_PER_BLK, 64), lambda i: (i, 0)),
    out_shape=jax.ShapeDtypeStruct((NVP * 4, 64), jnp.float32),
)


def _tout_body(y_ref, o_ref):
    o_ref[...] = y_ref[...].T


_tout = pl.pallas_call(
    _tout_body,
    grid=(NVP // 512,),
    in_specs=[pl.BlockSpec((512, BC), lambda i: (i, 0))],
    out_specs=pl.BlockSpec((BC, 512), lambda i: (0, i)),
    out_shape=jax.ShapeDtypeStruct((BC, NVK), jnp.float32),
)


def kernel(input, G_vals, L_vals, F2V_vals, NS, EW, coeffs,
           G_rows, G_cols, L_rows, L_cols, F2V_rows, F2V_cols):
    pad = NVP - NVK
    idx9 = G_cols.reshape(3, NFK, 3).transpose(1, 0, 2).reshape(-1)
    gv9 = G_vals.reshape(3, NFK, 3).transpose(1, 0, 2).reshape(NFK, 9)
    wma = jnp.concatenate([gv9, EW, NS, jnp.zeros((NFK, 1), jnp.float32)], axis=1)
    li7 = jnp.pad(L_cols.reshape(NVK, 7), ((0, pad), (0, 0))).reshape(-1)
    fi6 = jnp.pad(F2V_cols.reshape(NVK, 6), ((0, pad), (0, 0))).reshape(-1)
    wmb = jnp.pad(
        jnp.concatenate([L_vals.reshape(NVK, 7), F2V_vals.reshape(NVK, 6),
                         jnp.zeros((NVK, 3), jnp.float32)], axis=1),
        ((0, pad), (0, 0)))
    cs = jnp.stack([coeffs[k::4] for k in range(4)])

    xtp = _tin(input.reshape(BC, NVK))
    gf = _gf_kernel(xtp, idx9, wma)
    lap, gve, gvn = _lv_kernel(xtp, gf, li7, fi6, wmb)

    y4 = _mm(xtp.reshape(NVP * 4, 64), lap.reshape(NVP * 4, 64),
             gve.reshape(NVP * 4, 64), gvn.reshape(NVP * 4, 64), cs)
    return _tout(y4.reshape(NVP, BC)).reshape(4, 64, NVK)
